# Initial kernel scaffold; baseline (speedup 1.0000x reference)
import jax
import jax.numpy as jnp
from jax.experimental import pallas as pl


def kernel(x, W0, W1, W2, W3):
    raise NotImplementedError("write your pallas kernel here")


# speedup vs baseline: 11.3871x; 11.3871x over previous
import functools

import jax
import jax.numpy as jnp
from jax import lax
from jax.experimental import pallas as pl
from jax.experimental.pallas import tpu as pltpu
from jax.experimental.pallas import tpu_sc as plsc

B = 1024
T = 200
DIM = 128
VOCAB = 1000

NC = 2
NS = 16
L = 16
NW = NC * NS

N_ROWS = B * T
ROWS_PER_W = N_ROWS // NW
CHUNK = 128
N_CHUNKS = ROWS_PER_W // CHUNK
VECS_PER_CHUNK = CHUNK // L


def _gather_body(idx_hbm, table_hbm, out_hbm, raw_v, idx_v, row0_v, row1_v,
                 gsem, wsem):
    wid = lax.axis_index("s") * NC + lax.axis_index("c")
    base = wid * ROWS_PER_W

    pltpu.sync_copy(idx_hbm.at[pl.ds(base, ROWS_PER_W)], raw_v)

    off = (lax.iota(jnp.int32, L) % 4) * VOCAB

    def idx_body(c, carry):
        for j in range(VECS_PER_CHUNK):
            v = raw_v[pl.ds(c * CHUNK + j * L, L)] + off
            idx_v[c, pl.ds(j * L, L)] = v
        return carry

    lax.fori_loop(0, N_CHUNKS, idx_body, 0, unroll=False)

    rows = [row0_v, row1_v]

    pltpu.async_copy(table_hbm.at[idx_v.at[0]], row0_v, gsem)

    def chunk_body(c, carry):
        for p in range(2):
            cur = rows[p]
            nxt = rows[1 - p]
            chunk_id = c * 2 + p
            @pl.when(chunk_id + 1 < N_CHUNKS)
            def _():
                pltpu.async_copy(table_hbm.at[idx_v.at[chunk_id + 1]], nxt,
                                 gsem)
            pltpu.make_async_copy(table_hbm.at[idx_v.at[chunk_id]], cur,
                                  gsem).wait()
            cp = pltpu.async_copy(
                cur, out_hbm.at[pl.ds(base + chunk_id * CHUNK, CHUNK)], wsem)
            cp.wait()
        return carry

    lax.fori_loop(0, N_CHUNKS // 2, chunk_body, 0, unroll=False)


@jax.jit
def _rvq_embed(idx_flat, table):
    mesh = plsc.VectorSubcoreMesh(core_axis_name="c", subcore_axis_name="s")
    run = functools.partial(
        pl.kernel,
        out_type=jax.ShapeDtypeStruct((N_ROWS, DIM), jnp.float32),
        mesh=mesh,
        scratch_types=[
            pltpu.VMEM((ROWS_PER_W,), jnp.int32),
            pltpu.VMEM((N_CHUNKS, CHUNK), jnp.int32),
            pltpu.VMEM((CHUNK, DIM), jnp.float32),
            pltpu.VMEM((CHUNK, DIM), jnp.float32),
            pltpu.SemaphoreType.DMA,
            pltpu.SemaphoreType.DMA,
        ],
    )(_gather_body)
    return run(idx_flat, table)


def kernel(x, W0, W1, W2, W3):
    table = jnp.concatenate([W3, W0, W1, W2], axis=0)
    idx_flat = x.reshape(-1).astype(jnp.int32)
    out = _rvq_embed(idx_flat, table)
    return out.reshape(B, T, DIM)


# speedup vs baseline: 11.6133x; 1.0199x over previous
import functools

import jax
import jax.numpy as jnp
from jax import lax
from jax.experimental import pallas as pl
from jax.experimental.pallas import tpu as pltpu
from jax.experimental.pallas import tpu_sc as plsc

B = 1024
T = 200
DIM = 128
VOCAB = 1000

NC = 2
NS = 16
L = 16
NW = NC * NS

N_ROWS = B * T
ROWS_PER_W = N_ROWS // NW
CHUNK = 128
N_CHUNKS = ROWS_PER_W // CHUNK
VECS_PER_CHUNK = CHUNK // L
NBUF = 5
AHEAD = 3
N_GROUPS = N_CHUNKS // NBUF


def _gather_body(idx_hbm, table_hbm, out_hbm, raw_v, idx_v,
                 b0, b1, b2, b3, b4,
                 g0, g1, g2, g3, g4, w0, w1, w2, w3, w4):
    bufs = [b0, b1, b2, b3, b4]
    gsems = [g0, g1, g2, g3, g4]
    wsems = [w0, w1, w2, w3, w4]

    wid = lax.axis_index("s") * NC + lax.axis_index("c")
    base = wid * ROWS_PER_W

    pltpu.sync_copy(idx_hbm.at[pl.ds(base, ROWS_PER_W)], raw_v)

    off = (lax.iota(jnp.int32, L) % 4) * VOCAB

    def compute_idx(k):
        for j in range(VECS_PER_CHUNK):
            idx_v[k % N_CHUNKS, pl.ds(j * L, L)] = (
                raw_v[pl.ds((k % N_CHUNKS) * CHUNK + j * L, L)] + off)

    def fire_gather(k, p):
        pltpu.async_copy(table_hbm.at[idx_v.at[k % N_CHUNKS]], bufs[p],
                         gsems[p])

    def wait_gather(k, p):
        pltpu.make_async_copy(table_hbm.at[idx_v.at[k % N_CHUNKS]], bufs[p],
                              gsems[p]).wait()

    def fire_write(k, p):
        pltpu.async_copy(bufs[p],
                         out_hbm.at[pl.ds(base + (k % N_CHUNKS) * CHUNK,
                                          CHUNK)], wsems[p])

    def wait_write(p):
        pltpu.make_async_copy(bufs[p], out_hbm.at[pl.ds(base, CHUNK)],
                              wsems[p]).wait()

    for k in range(AHEAD):
        compute_idx(k)
        fire_gather(k, k)

    def group_body(g, carry):
        for p in range(NBUF):
            k = g * NBUF + p
            fp = (p + AHEAD) % NBUF

            @pl.when(jnp.logical_and(k + AHEAD >= NBUF,
                                     k + AHEAD < N_CHUNKS))
            def _():
                wait_write(fp)

            @pl.when(k + AHEAD < N_CHUNKS)
            def _():
                compute_idx(k + AHEAD)
                fire_gather(k + AHEAD, fp)

            wait_gather(k, p)
            fire_write(k, p)
        return carry

    lax.fori_loop(0, N_GROUPS, group_body, 0, unroll=False)

    for p in range(NBUF):
        wait_write(p)


@jax.jit
def _rvq_embed(idx_flat, table):
    mesh = plsc.VectorSubcoreMesh(core_axis_name="c", subcore_axis_name="s")
    run = functools.partial(
        pl.kernel,
        out_type=jax.ShapeDtypeStruct((N_ROWS, DIM), jnp.float32),
        mesh=mesh,
        scratch_types=[
            pltpu.VMEM((ROWS_PER_W,), jnp.int32),
            pltpu.VMEM((N_CHUNKS, CHUNK), jnp.int32),
        ] + [pltpu.VMEM((CHUNK, DIM), jnp.float32) for _ in range(NBUF)]
          + [pltpu.SemaphoreType.DMA for _ in range(2 * NBUF)],
    )(_gather_body)
    return run(idx_flat, table)


def kernel(x, W0, W1, W2, W3):
    table = jnp.concatenate([W3, W0, W1, W2], axis=0)
    idx_flat = x.reshape(-1).astype(jnp.int32)
    out = _rvq_embed(idx_flat, table)
    return out.reshape(B, T, DIM)


# speedup vs baseline: 18.2885x; 1.5748x over previous
import functools

import jax
import jax.numpy as jnp
from jax import lax
from jax.experimental import pallas as pl
from jax.experimental.pallas import tpu as pltpu
from jax.experimental.pallas import tpu_sc as plsc

B = 1024
T = 200
DIM = 128
VOCAB = 1000

NC = 2
NS = 16
L = 16
NW = NC * NS

N_ROWS = B * T
ROWS_PER_W = N_ROWS // NW
CHUNK = 128
N_CHUNKS = ROWS_PER_W // CHUNK
VECS_PER_CHUNK = CHUNK // L
NBUF = 5
AHEAD = 3
N_GROUPS = N_CHUNKS // NBUF


def _gather_body(idx_hbm, table_hbm, out_hbm, raw_v, idx_v, table_sp,
                 b0, b1, b2, b3, b4,
                 g0, g1, g2, g3, g4, w0, w1, w2, w3, w4):
    bufs = [b0, b1, b2, b3, b4]
    gsems = [g0, g1, g2, g3, g4]
    wsems = [w0, w1, w2, w3, w4]

    sid = lax.axis_index("s")
    wid = sid * NC + lax.axis_index("c")
    base = wid * ROWS_PER_W

    t_rows = 248
    pltpu.sync_copy(table_hbm.at[pl.ds(sid * t_rows, t_rows)],
                    table_sp.at[pl.ds(sid * t_rows, t_rows)])

    @pl.when(sid == 0)
    def _():
        pltpu.sync_copy(table_hbm.at[pl.ds(NS * t_rows, 4 * VOCAB - NS * t_rows)],
                        table_sp.at[pl.ds(NS * t_rows, 4 * VOCAB - NS * t_rows)])

    pltpu.sync_copy(idx_hbm.at[pl.ds(base, ROWS_PER_W)], raw_v)
    plsc.subcore_barrier()

    off = (lax.iota(jnp.int32, L) % 4) * VOCAB

    def compute_idx(k):
        for j in range(VECS_PER_CHUNK):
            idx_v[k % N_CHUNKS, pl.ds(j * L, L)] = (
                raw_v[pl.ds((k % N_CHUNKS) * CHUNK + j * L, L)] + off)

    def fire_gather(k, p):
        pltpu.async_copy(table_sp.at[idx_v.at[k % N_CHUNKS]], bufs[p],
                         gsems[p])

    def wait_gather(k, p):
        pltpu.make_async_copy(table_sp.at[idx_v.at[k % N_CHUNKS]], bufs[p],
                              gsems[p]).wait()

    def fire_write(k, p):
        pltpu.async_copy(bufs[p],
                         out_hbm.at[pl.ds(base + (k % N_CHUNKS) * CHUNK,
                                          CHUNK)], wsems[p])

    def wait_write(p):
        pltpu.make_async_copy(bufs[p], out_hbm.at[pl.ds(base, CHUNK)],
                              wsems[p]).wait()

    for k in range(AHEAD):
        compute_idx(k)
        fire_gather(k, k)

    def group_body(g, carry):
        for p in range(NBUF):
            k = g * NBUF + p
            fp = (p + AHEAD) % NBUF

            @pl.when(jnp.logical_and(k + AHEAD >= NBUF,
                                     k + AHEAD < N_CHUNKS))
            def _():
                wait_write(fp)

            @pl.when(k + AHEAD < N_CHUNKS)
            def _():
                compute_idx(k + AHEAD)
                fire_gather(k + AHEAD, fp)

            wait_gather(k, p)
            fire_write(k, p)
        return carry

    lax.fori_loop(0, N_GROUPS, group_body, 0, unroll=False)

    for p in range(NBUF):
        wait_write(p)


@jax.jit
def _rvq_embed(idx_flat, table):
    mesh = plsc.VectorSubcoreMesh(core_axis_name="c", subcore_axis_name="s")
    run = functools.partial(
        pl.kernel,
        out_type=jax.ShapeDtypeStruct((N_ROWS, DIM), jnp.float32),
        mesh=mesh,
        scratch_types=[
            pltpu.VMEM((ROWS_PER_W,), jnp.int32),
            pltpu.VMEM((N_CHUNKS, CHUNK), jnp.int32),
            pltpu.VMEM_SHARED((4 * VOCAB, DIM), jnp.float32),
        ] + [pltpu.VMEM((CHUNK, DIM), jnp.float32) for _ in range(NBUF)]
          + [pltpu.SemaphoreType.DMA for _ in range(2 * NBUF)],
    )(_gather_body)
    return run(idx_flat, table)


def kernel(x, W0, W1, W2, W3):
    table = jnp.concatenate([W3, W0, W1, W2], axis=0)
    idx_flat = x.reshape(-1).astype(jnp.int32)
    out = _rvq_embed(idx_flat, table)
    return out.reshape(B, T, DIM)


# speedup vs baseline: 18.4314x; 1.0078x over previous
import functools

import jax
import jax.numpy as jnp
from jax import lax
from jax.experimental import pallas as pl
from jax.experimental.pallas import tpu as pltpu
from jax.experimental.pallas import tpu_sc as plsc

B = 1024
T = 200
DIM = 128
VOCAB = 1000

NC = 2
NS = 16
L = 16
NW = NC * NS

N_ROWS = B * T
ROWS_PER_W = N_ROWS // NW
CHUNK = 128
N_CHUNKS = ROWS_PER_W // CHUNK
VECS_PER_CHUNK = CHUNK // L
NBUF = 5
AHEAD = 3
N_GROUPS = N_CHUNKS // NBUF
STAGE_ROWS = 64


def _gather_body(idx_hbm, t0_hbm, t1_hbm, t2_hbm, t3_hbm, out_hbm,
                 raw_v, idx_v, table_sp,
                 b0, b1, b2, b3, b4,
                 g0, g1, g2, g3, g4, w0, w1, w2, w3, w4):
    bufs = [b0, b1, b2, b3, b4]
    gsems = [g0, g1, g2, g3, g4]
    wsems = [w0, w1, w2, w3, w4]
    tables = [t0_hbm, t1_hbm, t2_hbm, t3_hbm]

    sid = lax.axis_index("s")
    wid = sid * NC + lax.axis_index("c")
    base = wid * ROWS_PER_W

    @pl.when(sid < NS - 1)
    def _():
        for q, t in enumerate(tables):
            pltpu.sync_copy(
                t.at[pl.ds(sid * STAGE_ROWS, STAGE_ROWS)],
                table_sp.at[pl.ds(q * VOCAB + sid * STAGE_ROWS, STAGE_ROWS)])

    @pl.when(sid == NS - 1)
    def _():
        rem = VOCAB - (NS - 1) * STAGE_ROWS
        for q, t in enumerate(tables):
            pltpu.sync_copy(
                t.at[pl.ds((NS - 1) * STAGE_ROWS, rem)],
                table_sp.at[pl.ds(q * VOCAB + (NS - 1) * STAGE_ROWS, rem)])

    pltpu.sync_copy(idx_hbm.at[pl.ds(base, ROWS_PER_W)], raw_v)
    plsc.subcore_barrier()

    off = (lax.iota(jnp.int32, L) % 4) * VOCAB

    def compute_idx(k):
        for j in range(VECS_PER_CHUNK):
            idx_v[k % N_CHUNKS, pl.ds(j * L, L)] = (
                raw_v[pl.ds((k % N_CHUNKS) * CHUNK + j * L, L)] + off)

    def fire_gather(k, p):
        pltpu.async_copy(table_sp.at[idx_v.at[k % N_CHUNKS]], bufs[p],
                         gsems[p])

    def wait_gather(k, p):
        pltpu.make_async_copy(table_sp.at[idx_v.at[k % N_CHUNKS]], bufs[p],
                              gsems[p]).wait()

    def fire_write(k, p):
        pltpu.async_copy(bufs[p],
                         out_hbm.at[pl.ds(base + (k % N_CHUNKS) * CHUNK,
                                          CHUNK)], wsems[p])

    def wait_write(p):
        pltpu.make_async_copy(bufs[p], out_hbm.at[pl.ds(base, CHUNK)],
                              wsems[p]).wait()

    for k in range(AHEAD):
        compute_idx(k)
        fire_gather(k, k)

    def group_body(g, carry):
        for p in range(NBUF):
            k = g * NBUF + p
            fp = (p + AHEAD) % NBUF

            @pl.when(jnp.logical_and(k + AHEAD >= NBUF,
                                     k + AHEAD < N_CHUNKS))
            def _():
                wait_write(fp)

            @pl.when(k + AHEAD < N_CHUNKS)
            def _():
                compute_idx(k + AHEAD)
                fire_gather(k + AHEAD, fp)

            wait_gather(k, p)
            fire_write(k, p)
        return carry

    lax.fori_loop(0, N_GROUPS, group_body, 0, unroll=False)

    for p in range(NBUF):
        wait_write(p)


@jax.jit
def _rvq_embed(idx_flat, t0, t1, t2, t3):
    mesh = plsc.VectorSubcoreMesh(core_axis_name="c", subcore_axis_name="s")
    run = functools.partial(
        pl.kernel,
        out_type=jax.ShapeDtypeStruct((N_ROWS, DIM), jnp.float32),
        mesh=mesh,
        scratch_types=[
            pltpu.VMEM((ROWS_PER_W,), jnp.int32),
            pltpu.VMEM((N_CHUNKS, CHUNK), jnp.int32),
            pltpu.VMEM_SHARED((4 * VOCAB, DIM), jnp.float32),
        ] + [pltpu.VMEM((CHUNK, DIM), jnp.float32) for _ in range(NBUF)]
          + [pltpu.SemaphoreType.DMA for _ in range(2 * NBUF)],
    )(_gather_body)
    return run(idx_flat, t0, t1, t2, t3)


def kernel(x, W0, W1, W2, W3):
    idx_flat = x.reshape(-1).astype(jnp.int32)
    out = _rvq_embed(idx_flat, W3, W0, W1, W2)
    return out.reshape(B, T, DIM)
